# baseline (speedup 1.0000x reference)
import functools

import jax
import jax.numpy as jnp
from jax import lax
from jax.experimental import pallas as pl
from jax.experimental.pallas import tpu as pltpu
from jax.experimental.pallas import tpu_sc as plsc

_B = 1024
_M = 100000
_D = 64
_DP = 128
_INV_TEMP = 20.0
_MOM = 0.2

_TN = 5120
_STRIP = 128

_NC = 2
_NS = 16
_NW = _NC * _NS
_BPW = _B // _NW


def _winner_body(col_ref, row_ref, w_ref):
    c = col_ref[...]
    r = row_ref[...]
    eq = c == r
    jj = lax.broadcasted_iota(jnp.int32, (_STRIP, _B), 1)
    w_ref[...] = jnp.max(jnp.where(eq, jj, -1), axis=1, keepdims=True)


def _winner(idx):
    col = idx.reshape(_B, 1)
    row = idx.reshape(1, _B)
    w = pl.pallas_call(
        _winner_body,
        grid=(_B // _STRIP,),
        in_specs=[
            pl.BlockSpec((_STRIP, 1), lambda s: (s, 0)),
            pl.BlockSpec((1, _B), lambda s: (0, 0)),
        ],
        out_specs=pl.BlockSpec((_STRIP, 1), lambda s: (s, 0)),
        out_shape=jax.ShapeDtypeStruct((_B, 1), jnp.int32),
    )(col, row)
    return w.reshape(_B)


def _mm_body(xt_ref, ft_ref, o_ref, nf_ref):
    f = ft_ref[...]
    o_ref[...] = lax.dot_general(
        f, xt_ref[...], (((0,), (0,)), ((), ())),
        preferred_element_type=jnp.float32,
    ) * _INV_TEMP
    nf_ref[...] = jnp.concatenate(
        [f.T, jnp.zeros((_TN, _DP - _D), jnp.float32)], axis=1)


def _matmul(xt, ft):
    return pl.pallas_call(
        _mm_body,
        grid=(pl.cdiv(_M, _TN),),
        in_specs=[
            pl.BlockSpec((_D, _B), lambda n: (0, 0)),
            pl.BlockSpec((_D, _TN), lambda n: (0, n)),
        ],
        out_specs=[
            pl.BlockSpec((_TN, _B), lambda n: (n, 0)),
            pl.BlockSpec((_TN, _DP), lambda n: (n, 0)),
        ],
        out_shape=[
            jax.ShapeDtypeStruct((_M, _B), jnp.float32),
            jax.ShapeDtypeStruct((_M, _DP), jnp.float32),
        ],
        compiler_params=pltpu.CompilerParams(
            dimension_semantics=("parallel",),
        ),
    )(xt, ft)


def _sc_update_body(bank, inp, idx_h, w_h, stage,
                    idx_v, widx_v, old_v, in_v, u_v, sem):
    wid = lax.axis_index("s") * _NC + lax.axis_index("c")
    base = wid * _BPW
    pltpu.sync_copy(idx_h.at[pl.ds(base, _BPW)], idx_v)
    pltpu.sync_copy(w_h.at[pl.ds(base, _BPW)], widx_v)
    pltpu.async_copy(bank.at[idx_v], old_v, sem).wait()
    pltpu.async_copy(inp.at[widx_v], in_v, sem).wait()
    nchunk = _D // 16
    for r in range(_BPW):
        us = []
        ssq = jnp.zeros((16,), jnp.float32)
        for c in range(nchunk):
            o = old_v[r, pl.ds(c * 16, 16)]
            x = in_v[r, pl.ds(c * 16, 16)]
            u = _MOM * o + (1.0 - _MOM) * x
            us.append(u)
            ssq = ssq + u * u
        s = jnp.broadcast_to(jnp.sum(ssq), (16,))
        bits = lax.bitcast_convert_type(s, jnp.int32)
        y = lax.bitcast_convert_type(
            jnp.int32(0x5F3759DF) - (bits >> 1), jnp.float32)
        for _ in range(3):
            y = y * (1.5 - 0.5 * s * y * y)
        for c in range(nchunk):
            u_v[r, pl.ds(c * 16, 16)] = us[c] * y
        zero = jnp.zeros((16,), jnp.float32)
        for c in range(nchunk, _DP // 16):
            u_v[r, pl.ds(c * 16, 16)] = zero
    pltpu.sync_copy(u_v, stage.at[pl.ds(base, _BPW)])


def _sc_scatter_body(stage, idx_h, nf, idx_v, row_v, sem):
    wid = lax.axis_index("s") * _NC + lax.axis_index("c")
    base = wid * _BPW
    pltpu.sync_copy(idx_h.at[pl.ds(base, _BPW)], idx_v)
    pltpu.sync_copy(stage.at[pl.ds(base, _BPW)], row_v)
    pltpu.async_copy(row_v, nf.at[idx_v], sem).wait()


@functools.cache
def _get_sc_kernels():
    mesh = plsc.VectorSubcoreMesh(
        core_axis_name="c", subcore_axis_name="s",
        num_cores=_NC, num_subcores=_NS)
    params = pltpu.CompilerParams(needs_layout_passes=False)
    update = pl.kernel(
        _sc_update_body,
        out_type=jax.ShapeDtypeStruct((_B, _DP), jnp.float32),
        mesh=mesh,
        compiler_params=params,
        scratch_types=[
            pltpu.VMEM((_BPW,), jnp.int32),
            pltpu.VMEM((_BPW,), jnp.int32),
            pltpu.VMEM((_BPW, _DP), jnp.float32),
            pltpu.VMEM((_BPW, _DP), jnp.float32),
            pltpu.VMEM((_BPW, _DP), jnp.float32),
            pltpu.SemaphoreType.DMA,
        ],
    )
    scatter = pl.kernel(
        _sc_scatter_body,
        out_type=(),
        mesh=mesh,
        compiler_params=params,
        scratch_types=[
            pltpu.VMEM((_BPW,), jnp.int32),
            pltpu.VMEM((_BPW, _DP), jnp.float32),
            pltpu.SemaphoreType.DMA,
        ],
    )
    return update, scatter


def kernel(inputs, indexes, features):
    idx = indexes.astype(jnp.int32)
    w = _winner(idx)
    out_t, nf0 = _matmul(inputs.T, features.T)
    inputs_p = jnp.pad(inputs, ((0, 0), (0, _DP - _D)))
    update, scatter = _get_sc_kernels()
    nf_ref = jax.new_ref(nf0)
    stage = update(nf_ref, inputs_p, idx, w)
    scatter(stage, idx, nf_ref)
    return out_t.T, nf_ref[...][:, :_D]


# speedup vs baseline: 1.0061x; 1.0061x over previous
import functools

import jax
import jax.numpy as jnp
from jax import lax
from jax.experimental import pallas as pl
from jax.experimental.pallas import tpu as pltpu
from jax.experimental.pallas import tpu_sc as plsc

_B = 1024
_M = 100000
_D = 64
_DP = 128
_INV_TEMP = 20.0
_MOM = 0.2

_TN = 4096
_STRIP = 128

_NC = 2
_NS = 16
_NW = _NC * _NS
_BPW = _B // _NW


def _winner_body(col_ref, row_ref, w_ref):
    c = col_ref[...]
    r = row_ref[...]
    eq = c == r
    jj = lax.broadcasted_iota(jnp.int32, (_STRIP, _B), 1)
    w_ref[...] = jnp.max(jnp.where(eq, jj, -1), axis=1, keepdims=True)


def _winner(idx):
    col = idx.reshape(_B, 1)
    row = idx.reshape(1, _B)
    w = pl.pallas_call(
        _winner_body,
        grid=(_B // _STRIP,),
        in_specs=[
            pl.BlockSpec((_STRIP, 1), lambda s: (s, 0)),
            pl.BlockSpec((1, _B), lambda s: (0, 0)),
        ],
        out_specs=pl.BlockSpec((_STRIP, 1), lambda s: (s, 0)),
        out_shape=jax.ShapeDtypeStruct((_B, 1), jnp.int32),
    )(col, row)
    return w.reshape(_B)


def _mm_body(xt_ref, ft_ref, o_ref, nf_ref):
    f = ft_ref[...]
    o_ref[...] = lax.dot_general(
        f, xt_ref[...], (((0,), (0,)), ((), ())),
        preferred_element_type=jnp.float32,
    ) * _INV_TEMP
    nf_ref[...] = jnp.concatenate(
        [f.T, jnp.zeros((_TN, _DP - _D), jnp.float32)], axis=1)


def _matmul(xt, ft):
    return pl.pallas_call(
        _mm_body,
        grid=(pl.cdiv(_M, _TN),),
        in_specs=[
            pl.BlockSpec((_D, _B), lambda n: (0, 0)),
            pl.BlockSpec((_D, _TN), lambda n: (0, n)),
        ],
        out_specs=[
            pl.BlockSpec((_TN, _B), lambda n: (n, 0)),
            pl.BlockSpec((_TN, _DP), lambda n: (n, 0)),
        ],
        out_shape=[
            jax.ShapeDtypeStruct((_M, _B), jnp.float32),
            jax.ShapeDtypeStruct((_M, _DP), jnp.float32),
        ],
        compiler_params=pltpu.CompilerParams(
            dimension_semantics=("parallel",),
        ),
    )(xt, ft)


def _sc_update_body(bank, inp, idx_h, w_h, stage,
                    idx_v, widx_v, old_v, in_v, u_v, sem):
    wid = lax.axis_index("s") * _NC + lax.axis_index("c")
    base = wid * _BPW
    pltpu.sync_copy(idx_h.at[pl.ds(base, _BPW)], idx_v)
    pltpu.sync_copy(w_h.at[pl.ds(base, _BPW)], widx_v)
    pltpu.async_copy(bank.at[idx_v], old_v, sem).wait()
    pltpu.async_copy(inp.at[widx_v], in_v, sem).wait()
    nchunk = _D // 16
    for r in range(_BPW):
        us = []
        ssq = jnp.zeros((16,), jnp.float32)
        for c in range(nchunk):
            o = old_v[r, pl.ds(c * 16, 16)]
            x = in_v[r, pl.ds(c * 16, 16)]
            u = _MOM * o + (1.0 - _MOM) * x
            us.append(u)
            ssq = ssq + u * u
        s = jnp.broadcast_to(jnp.sum(ssq), (16,))
        bits = lax.bitcast_convert_type(s, jnp.int32)
        y = lax.bitcast_convert_type(
            jnp.int32(0x5F3759DF) - (bits >> 1), jnp.float32)
        for _ in range(3):
            y = y * (1.5 - 0.5 * s * y * y)
        for c in range(nchunk):
            u_v[r, pl.ds(c * 16, 16)] = us[c] * y
        zero = jnp.zeros((16,), jnp.float32)
        for c in range(nchunk, _DP // 16):
            u_v[r, pl.ds(c * 16, 16)] = zero
    pltpu.sync_copy(u_v, stage.at[pl.ds(base, _BPW)])


def _sc_scatter_body(stage, idx_h, nf, idx_v, row_v, sem):
    wid = lax.axis_index("s") * _NC + lax.axis_index("c")
    base = wid * _BPW
    pltpu.sync_copy(idx_h.at[pl.ds(base, _BPW)], idx_v)
    pltpu.sync_copy(stage.at[pl.ds(base, _BPW)], row_v)
    pltpu.async_copy(row_v, nf.at[idx_v], sem).wait()


@functools.cache
def _get_sc_kernels():
    mesh = plsc.VectorSubcoreMesh(
        core_axis_name="c", subcore_axis_name="s",
        num_cores=_NC, num_subcores=_NS)
    params = pltpu.CompilerParams(needs_layout_passes=False)
    update = pl.kernel(
        _sc_update_body,
        out_type=jax.ShapeDtypeStruct((_B, _DP), jnp.float32),
        mesh=mesh,
        compiler_params=params,
        scratch_types=[
            pltpu.VMEM((_BPW,), jnp.int32),
            pltpu.VMEM((_BPW,), jnp.int32),
            pltpu.VMEM((_BPW, _DP), jnp.float32),
            pltpu.VMEM((_BPW, _DP), jnp.float32),
            pltpu.VMEM((_BPW, _DP), jnp.float32),
            pltpu.SemaphoreType.DMA,
        ],
    )
    scatter = pl.kernel(
        _sc_scatter_body,
        out_type=(),
        mesh=mesh,
        compiler_params=params,
        scratch_types=[
            pltpu.VMEM((_BPW,), jnp.int32),
            pltpu.VMEM((_BPW, _DP), jnp.float32),
            pltpu.SemaphoreType.DMA,
        ],
    )
    return update, scatter


def kernel(inputs, indexes, features):
    idx = indexes.astype(jnp.int32)
    w = _winner(idx)
    out_t, nf0 = _matmul(inputs.T, features.T)
    inputs_p = jnp.pad(inputs, ((0, 0), (0, _DP - _D)))
    update, scatter = _get_sc_kernels()
    nf_ref = jax.new_ref(nf0)
    stage = update(nf_ref, inputs_p, idx, w)
    scatter(stage, idx, nf_ref)
    return out_t.T, nf_ref[...][:, :_D]
